# Initial kernel scaffold; baseline (speedup 1.0000x reference)
import jax
import jax.numpy as jnp
from jax.experimental import pallas as pl


def kernel(log_w, particles, observation, A, Ap, b, C, proc_log_scale, prop_log_scale, obs_log_scale):
    raise NotImplementedError("write your pallas kernel here")


# speedup vs baseline: 6.1283x; 6.1283x over previous
import functools

import numpy as np

import jax
import jax.numpy as jnp
from jax import lax
from jax.experimental import pallas as pl
from jax.experimental.pallas import tpu as pltpu
from jax.experimental.pallas import tpu_sc as plsc

_N = 16384
_D = 128
_LOG2PI = float(np.log(2.0 * np.pi))

_NW = 32
_BPW = _N // _NW
_CHUNK = 128
_NCHUNK = _BPW // _CHUNK


def _rotl(x, r):
    return ((x << np.uint32(r)) | (x >> np.uint32(32 - r))).astype(np.uint32)


def _threefry2x32(k0, k1, x0, x1):
    k0 = np.uint32(k0)
    k1 = np.uint32(k1)
    ks2 = np.uint32(k0 ^ k1 ^ np.uint32(0x1BD11BDA))
    x0 = x0.astype(np.uint32).copy()
    x1 = x1.astype(np.uint32).copy()
    rot1 = (13, 15, 26, 6)
    rot2 = (17, 29, 16, 24)
    x0 += k0
    x1 += k1
    for r in rot1:
        x0 += x1
        x1 = _rotl(x1, r)
        x1 ^= x0
    x0 += k1
    x1 += ks2 + np.uint32(1)
    for r in rot2:
        x0 += x1
        x1 = _rotl(x1, r)
        x1 ^= x0
    x0 += ks2
    x1 += k0 + np.uint32(2)
    for r in rot1:
        x0 += x1
        x1 = _rotl(x1, r)
        x1 ^= x0
    x0 += k0
    x1 += k1 + np.uint32(3)
    for r in rot2:
        x0 += x1
        x1 = _rotl(x1, r)
        x1 ^= x0
    x0 += k1
    x1 += ks2 + np.uint32(4)
    for r in rot1:
        x0 += x1
        x1 = _rotl(x1, r)
        x1 ^= x0
    x0 += ks2
    x1 += k0 + np.uint32(5)
    return x0, x1


def _random_bits(k0, k1, n):
    i = np.arange(n, dtype=np.uint64)
    x0 = (i >> np.uint64(32)).astype(np.uint32)
    x1 = (i & np.uint64(0xFFFFFFFF)).astype(np.uint32)
    o0, o1 = _threefry2x32(k0, k1, x0, x1)
    return o0 ^ o1


def _bits_to_unit_float(bits):
    fb = (bits >> np.uint32(9)) | np.uint32(0x3F800000)
    return fb.view(np.float32) - np.float32(1.0)


def _erfinv64(x):
    w = -np.log((np.float64(1.0) - x) * (np.float64(1.0) + x))
    small = w < 5.0
    ws = w - 2.5
    p_s = np.float64(2.81022636e-08)
    for c in (3.43273939e-07, -3.5233877e-06, -4.39150654e-06, 0.00021858087,
              -0.00125372503, -0.00417768164, 0.246640727, 1.50140941):
        p_s = p_s * ws + c
    wl = np.sqrt(np.maximum(w, 1e-30)) - 3.0
    p_l = np.float64(-0.000200214257)
    for c in (0.000100950558, 0.00134934322, -0.00367342844, 0.00573950773,
              -0.0076224613, 0.00943887047, 1.00167406, 2.83297682):
        p_l = p_l * wl + c
    return np.where(small, p_s, p_l) * x


def _rng_constants():
    with np.errstate(over="ignore"):
        o0, o1 = _threefry2x32(np.uint32(0), np.uint32(42),
                               np.zeros(2, np.uint32),
                               np.arange(2, dtype=np.uint32))
        rk = (o0[0], o1[0])
        pk = (o0[1], o1[1])
        u0 = np.maximum(np.float32(0.0),
                        _bits_to_unit_float(_random_bits(rk[0], rk[1], 1)))[0]
        f = _bits_to_unit_float(_random_bits(pk[0], pk[1], _N * _D))
        lo = np.float32(np.nextafter(np.float32(-1.0), np.float32(0.0)))
        u = np.maximum(lo, (f * (np.float32(1.0) - lo) + lo).astype(np.float32))
        noise = (np.float64(np.sqrt(2.0))
                 * _erfinv64(u.astype(np.float64))).astype(np.float32)
    return np.float32(u0), noise.reshape(_N, _D)


_U0, _NOISE = _rng_constants()


def _sc_resample_body(cum_hbm, part_hbm, out_hbm, cum_v, idx_v, rows_v, sem):
    wid = lax.axis_index("s") * 2 + lax.axis_index("c")
    base = wid * _BPW
    pltpu.sync_copy(cum_hbm, cum_v)
    lane = lax.iota(jnp.int32, 16)
    u0 = jnp.float32(_U0)
    inv_n = jnp.float32(1.0 / _N)

    for j in range(_NCHUNK):
        row0 = base + j * _CHUNK

        def body(g, carry, row0=row0):
            col = g * 16
            i_vec = lane + (row0 + col)
            u = (i_vec.astype(jnp.float32) + u0) * inv_n
            lo = jnp.zeros((16,), jnp.int32)
            half = _N // 2
            while half >= 1:
                probe = lo + (half - 1)
                val = plsc.load_gather(cum_v, [probe])
                lo = lo + jnp.where(val < u, half, 0).astype(jnp.int32)
                half //= 2
            idx_v[j, pl.ds(col, 16)] = jnp.minimum(lo, _N - 1)
            return carry

        lax.fori_loop(0, _CHUNK // 16, body, 0)
        pltpu.async_copy(part_hbm.at[idx_v.at[j]], rows_v, sem).wait()
        pltpu.sync_copy(rows_v, out_hbm.at[pl.ds(row0, _CHUNK)])


def _sc_resample(cum, particles):
    mesh = plsc.VectorSubcoreMesh(core_axis_name="c", subcore_axis_name="s")
    k = functools.partial(
        pl.kernel,
        mesh=mesh,
        out_type=jax.ShapeDtypeStruct((_N, _D), jnp.float32),
        scratch_types=[
            pltpu.VMEM((_N,), jnp.float32),
            pltpu.VMEM((_NCHUNK, _CHUNK), jnp.int32),
            pltpu.VMEM((_CHUNK, _D), jnp.float32),
            pltpu.SemaphoreType.DMA,
        ],
        compiler_params=pltpu.CompilerParams(needs_layout_passes=False),
    )(_sc_resample_body)
    return k(cum, particles)


_BLK = 1024


def _dense_body(res_ref, noise_ref, logw_ref, lw_ref, obs_ref, A_ref, Ap_ref,
                b_ref, C_ref, pls_ref, qls_ref, ols_ref,
                nlw_ref, nxt_ref, ess_ref, acc_ref):
    i = pl.program_id(0)
    res = res_ref[...]
    noise = noise_ref[...]
    b = b_ref[...]
    pls = pls_ref[...]
    qls = qls_ref[...]
    ols = ols_ref[...]
    obs = obs_ref[...]

    dn = (((1,), (1,)), ((), ()))
    mean_p = lax.dot_general(res, Ap_ref[...], dn,
                             preferred_element_type=jnp.float32) + b
    nxt = mean_p + jnp.exp(qls) * noise
    mean_t = lax.dot_general(res, A_ref[...], dn,
                             preferred_element_type=jnp.float32) + b
    y = lax.dot_general(nxt, C_ref[...], dn,
                        preferred_element_type=jnp.float32)

    zt = (nxt - mean_t) * jnp.exp(-pls)
    zp = (nxt - mean_p) * jnp.exp(-qls)
    ze = (obs - y) * jnp.exp(-ols)

    half_d_log2pi = 0.5 * _D * _LOG2PI
    t_logp = jnp.sum(-0.5 * zt * zt, axis=1, keepdims=True) - (jnp.sum(pls) + half_d_log2pi)
    p_logp = jnp.sum(-0.5 * zp * zp, axis=1, keepdims=True) - (jnp.sum(qls) + half_d_log2pi)
    e_logp = jnp.sum(-0.5 * ze * ze, axis=1, keepdims=True) - (jnp.sum(ols) + half_d_log2pi)

    nlw_ref[...] = logw_ref[...] + (t_logp + e_logp - p_logp)
    nxt_ref[...] = nxt

    lw = lw_ref[...]
    part = jnp.sum(jnp.exp(lw + lw), keepdims=True)

    @pl.when(i == 0)
    def _():
        acc_ref[...] = part

    @pl.when(i > 0)
    def _():
        acc_ref[...] = acc_ref[...] + part

    @pl.when(i == pl.num_programs(0) - 1)
    def _():
        ess_ref[...] = 1.0 / (jnp.float32(_N) * acc_ref[...])


def _dense(res, lw, log_w, observation, A, Ap, b, C, pls, qls, ols):
    grid = (_N // _BLK,)
    row_blk = lambda i: (i, 0)
    const_blk = lambda i: (0, 0)
    out_shapes = (
        jax.ShapeDtypeStruct((_N, 1), jnp.float32),
        jax.ShapeDtypeStruct((_N, _D), jnp.float32),
        jax.ShapeDtypeStruct((1, 1), jnp.float32),
    )
    in_specs = [
        pl.BlockSpec((_BLK, _D), row_blk),
        pl.BlockSpec((_BLK, _D), row_blk),
        pl.BlockSpec((_BLK, 1), row_blk),
        pl.BlockSpec((_BLK, 1), row_blk),
        pl.BlockSpec((1, _D), const_blk),
        pl.BlockSpec((_D, _D), const_blk),
        pl.BlockSpec((_D, _D), const_blk),
        pl.BlockSpec((1, _D), const_blk),
        pl.BlockSpec((_D, _D), const_blk),
        pl.BlockSpec((1, _D), const_blk),
        pl.BlockSpec((1, _D), const_blk),
        pl.BlockSpec((1, _D), const_blk),
    ]
    out_specs = (
        pl.BlockSpec((_BLK, 1), row_blk),
        pl.BlockSpec((_BLK, _D), row_blk),
        pl.BlockSpec((1, 1), const_blk),
    )
    return pl.pallas_call(
        _dense_body,
        grid=grid,
        in_specs=in_specs,
        out_specs=out_specs,
        out_shape=out_shapes,
        scratch_shapes=[pltpu.VMEM((1, 1), jnp.float32)],
        compiler_params=pltpu.CompilerParams(
            dimension_semantics=("arbitrary",),
        ),
    )(res, _NOISE, log_w.reshape(_N, 1), lw.reshape(_N, 1),
      observation.reshape(1, _D), A, Ap, b.reshape(1, _D), C,
      pls.reshape(1, _D), qls.reshape(1, _D), ols.reshape(1, _D))


def kernel(log_w, particles, observation, A, Ap, b, C,
           proc_log_scale, prop_log_scale, obs_log_scale):
    lw = log_w - jax.scipy.special.logsumexp(log_w)
    w = jnp.exp(lw)
    cum = jnp.cumsum(w)

    res = _sc_resample(cum, particles)
    nlw, nxt, ess = _dense(res, lw, log_w, observation, A, Ap, b, C,
                           proc_log_scale, prop_log_scale, obs_log_scale)
    return nlw.reshape(_N), nxt, ess.reshape(())


# speedup vs baseline: 8.5616x; 1.3971x over previous
import functools

import numpy as np

import jax
import jax.numpy as jnp
from jax import lax
from jax.experimental import pallas as pl
from jax.experimental.pallas import tpu as pltpu
from jax.experimental.pallas import tpu_sc as plsc

_N = 16384
_D = 128
_LOG2PI = float(np.log(2.0 * np.pi))

_NW = 32
_BPW = _N // _NW
_CHUNK = 128
_NCHUNK = _BPW // _CHUNK


def _rotl(x, r):
    return ((x << np.uint32(r)) | (x >> np.uint32(32 - r))).astype(np.uint32)


def _threefry2x32(k0, k1, x0, x1):
    k0 = np.uint32(k0)
    k1 = np.uint32(k1)
    ks2 = np.uint32(k0 ^ k1 ^ np.uint32(0x1BD11BDA))
    x0 = x0.astype(np.uint32).copy()
    x1 = x1.astype(np.uint32).copy()
    rot1 = (13, 15, 26, 6)
    rot2 = (17, 29, 16, 24)
    x0 += k0
    x1 += k1
    for r in rot1:
        x0 += x1
        x1 = _rotl(x1, r)
        x1 ^= x0
    x0 += k1
    x1 += ks2 + np.uint32(1)
    for r in rot2:
        x0 += x1
        x1 = _rotl(x1, r)
        x1 ^= x0
    x0 += ks2
    x1 += k0 + np.uint32(2)
    for r in rot1:
        x0 += x1
        x1 = _rotl(x1, r)
        x1 ^= x0
    x0 += k0
    x1 += k1 + np.uint32(3)
    for r in rot2:
        x0 += x1
        x1 = _rotl(x1, r)
        x1 ^= x0
    x0 += k1
    x1 += ks2 + np.uint32(4)
    for r in rot1:
        x0 += x1
        x1 = _rotl(x1, r)
        x1 ^= x0
    x0 += ks2
    x1 += k0 + np.uint32(5)
    return x0, x1


def _random_bits(k0, k1, n):
    i = np.arange(n, dtype=np.uint64)
    x0 = (i >> np.uint64(32)).astype(np.uint32)
    x1 = (i & np.uint64(0xFFFFFFFF)).astype(np.uint32)
    o0, o1 = _threefry2x32(k0, k1, x0, x1)
    return o0 ^ o1


def _bits_to_unit_float(bits):
    fb = (bits >> np.uint32(9)) | np.uint32(0x3F800000)
    return fb.view(np.float32) - np.float32(1.0)


def _erfinv64(x):
    w = -np.log((np.float64(1.0) - x) * (np.float64(1.0) + x))
    small = w < 5.0
    ws = w - 2.5
    p_s = np.float64(2.81022636e-08)
    for c in (3.43273939e-07, -3.5233877e-06, -4.39150654e-06, 0.00021858087,
              -0.00125372503, -0.00417768164, 0.246640727, 1.50140941):
        p_s = p_s * ws + c
    wl = np.sqrt(np.maximum(w, 1e-30)) - 3.0
    p_l = np.float64(-0.000200214257)
    for c in (0.000100950558, 0.00134934322, -0.00367342844, 0.00573950773,
              -0.0076224613, 0.00943887047, 1.00167406, 2.83297682):
        p_l = p_l * wl + c
    return np.where(small, p_s, p_l) * x


def _rng_constants():
    with np.errstate(over="ignore"):
        o0, o1 = _threefry2x32(np.uint32(0), np.uint32(42),
                               np.zeros(2, np.uint32),
                               np.arange(2, dtype=np.uint32))
        rk = (o0[0], o1[0])
        pk = (o0[1], o1[1])
        u0 = np.maximum(np.float32(0.0),
                        _bits_to_unit_float(_random_bits(rk[0], rk[1], 1)))[0]
        f = _bits_to_unit_float(_random_bits(pk[0], pk[1], _N * _D))
        lo = np.float32(np.nextafter(np.float32(-1.0), np.float32(0.0)))
        u = np.maximum(lo, (f * (np.float32(1.0) - lo) + lo).astype(np.float32))
        noise = (np.float64(np.sqrt(2.0))
                 * _erfinv64(u.astype(np.float64))).astype(np.float32)
    return np.float32(u0), noise.reshape(_N, _D)


_U0, _NOISE = _rng_constants()


def _sc_resample_body(cum_hbm, part_hbm, out_hbm, cum_v, idx_v, rows_v,
                      gsems, ssem):
    wid = lax.axis_index("s") * 2 + lax.axis_index("c")
    base = wid * _BPW
    pltpu.sync_copy(cum_hbm, cum_v)
    lane = lax.iota(jnp.int32, 16)
    u0 = jnp.float32(_U0)
    inv_n = jnp.float32(1.0 / _N)

    def _rank16(i_vec):
        u = (i_vec.astype(jnp.float32) + u0) * inv_n
        lo = jnp.zeros((16,), jnp.int32)
        half = _N // 2
        while half >= 1:
            probe = lo + (half - 1)
            val = plsc.load_gather(cum_v, [probe])
            lo = lo + jnp.where(val < u, half, 0).astype(jnp.int32)
            half //= 2
        return jnp.minimum(lo, _N - 1)

    gathers = []
    for j in range(_NCHUNK):
        row0 = base + j * _CHUNK

        def body(g, carry, row0=row0, j=j):
            col = g * 32
            idx_v[j, pl.ds(col, 16)] = _rank16(lane + (row0 + col))
            idx_v[j, pl.ds(col + 16, 16)] = _rank16(lane + (row0 + col + 16))
            return carry

        lax.fori_loop(0, _CHUNK // 32, body, 0)
        gathers.append(
            pltpu.async_copy(part_hbm.at[idx_v.at[j]], rows_v.at[j], gsems[j]))
    stores = []
    for j in range(_NCHUNK):
        gathers[j].wait()
        stores.append(
            pltpu.async_copy(rows_v.at[j],
                             out_hbm.at[pl.ds(base + j * _CHUNK, _CHUNK)],
                             ssem))
    for s in stores:
        s.wait()


def _sc_resample(cum, particles):
    mesh = plsc.VectorSubcoreMesh(core_axis_name="c", subcore_axis_name="s")
    k = functools.partial(
        pl.kernel,
        mesh=mesh,
        out_type=jax.ShapeDtypeStruct((_N, _D), jnp.float32),
        scratch_types=[
            pltpu.VMEM((_N,), jnp.float32),
            pltpu.VMEM((_NCHUNK, _CHUNK), jnp.int32),
            pltpu.VMEM((_NCHUNK, _CHUNK, _D), jnp.float32),
            [pltpu.SemaphoreType.DMA] * _NCHUNK,
            pltpu.SemaphoreType.DMA,
        ],
        compiler_params=pltpu.CompilerParams(needs_layout_passes=False),
    )(_sc_resample_body)
    return k(cum, particles)


_BLK = 1024


def _dense_body(res_ref, noise_ref, logw_ref, lw_ref, obs_ref, A_ref, Ap_ref,
                b_ref, C_ref, pls_ref, qls_ref, ols_ref,
                nlw_ref, nxt_ref, ess_ref, acc_ref):
    i = pl.program_id(0)
    res = res_ref[...]
    noise = noise_ref[...]
    b = b_ref[...]
    pls = pls_ref[...]
    qls = qls_ref[...]
    ols = ols_ref[...]
    obs = obs_ref[...]

    dn = (((1,), (1,)), ((), ()))
    mean_p = lax.dot_general(res, Ap_ref[...], dn,
                             preferred_element_type=jnp.float32) + b
    nxt = mean_p + jnp.exp(qls) * noise
    mean_t = lax.dot_general(res, A_ref[...], dn,
                             preferred_element_type=jnp.float32) + b
    y = lax.dot_general(nxt, C_ref[...], dn,
                        preferred_element_type=jnp.float32)

    zt = (nxt - mean_t) * jnp.exp(-pls)
    zp = (nxt - mean_p) * jnp.exp(-qls)
    ze = (obs - y) * jnp.exp(-ols)

    half_d_log2pi = 0.5 * _D * _LOG2PI
    q = -0.5 * (zt * zt + ze * ze - zp * zp)
    const = jnp.sum(pls) + jnp.sum(ols) - jnp.sum(qls) + half_d_log2pi
    inc = (jnp.sum(q, axis=1, keepdims=True) - const).reshape(_BLK // _D, _D)
    nlw_ref[...] = logw_ref[...] + inc
    nxt_ref[...] = nxt

    lw = lw_ref[...]
    part = jnp.sum(jnp.exp(lw + lw), keepdims=True)

    @pl.when(i == 0)
    def _():
        acc_ref[...] = part

    @pl.when(i > 0)
    def _():
        acc_ref[...] = acc_ref[...] + part

    @pl.when(i == pl.num_programs(0) - 1)
    def _():
        ess_ref[...] = 1.0 / (jnp.float32(_N) * acc_ref[...])


def _dense(res, lw, log_w, observation, A, Ap, b, C, pls, qls, ols):
    grid = (_N // _BLK,)
    row_blk = lambda i: (i, 0)
    const_blk = lambda i: (0, 0)
    out_shapes = (
        jax.ShapeDtypeStruct((_N // _D, _D), jnp.float32),
        jax.ShapeDtypeStruct((_N, _D), jnp.float32),
        jax.ShapeDtypeStruct((1, 1), jnp.float32),
    )
    in_specs = [
        pl.BlockSpec((_BLK, _D), row_blk),
        pl.BlockSpec((_BLK, _D), row_blk),
        pl.BlockSpec((_BLK // _D, _D), row_blk),
        pl.BlockSpec((_BLK // _D, _D), row_blk),
        pl.BlockSpec((1, _D), const_blk),
        pl.BlockSpec((_D, _D), const_blk),
        pl.BlockSpec((_D, _D), const_blk),
        pl.BlockSpec((1, _D), const_blk),
        pl.BlockSpec((_D, _D), const_blk),
        pl.BlockSpec((1, _D), const_blk),
        pl.BlockSpec((1, _D), const_blk),
        pl.BlockSpec((1, _D), const_blk),
    ]
    out_specs = (
        pl.BlockSpec((_BLK // _D, _D), row_blk),
        pl.BlockSpec((_BLK, _D), row_blk),
        pl.BlockSpec((1, 1), const_blk),
    )
    return pl.pallas_call(
        _dense_body,
        grid=grid,
        in_specs=in_specs,
        out_specs=out_specs,
        out_shape=out_shapes,
        scratch_shapes=[pltpu.VMEM((1, 1), jnp.float32)],
        compiler_params=pltpu.CompilerParams(
            dimension_semantics=("arbitrary",),
        ),
    )(res, _NOISE, log_w.reshape(_N // _D, _D), lw.reshape(_N // _D, _D),
      observation.reshape(1, _D), A, Ap, b.reshape(1, _D), C,
      pls.reshape(1, _D), qls.reshape(1, _D), ols.reshape(1, _D))


def kernel(log_w, particles, observation, A, Ap, b, C,
           proc_log_scale, prop_log_scale, obs_log_scale):
    lw = log_w - jax.scipy.special.logsumexp(log_w)
    w = jnp.exp(lw)
    cum = jnp.cumsum(w)

    res = _sc_resample(cum, particles)
    nlw, nxt, ess = _dense(res, lw, log_w, observation, A, Ap, b, C,
                           proc_log_scale, prop_log_scale, obs_log_scale)
    return nlw.reshape(_N), nxt, ess.reshape(())


# speedup vs baseline: 9.3703x; 1.0945x over previous
import functools

import numpy as np

import jax
import jax.numpy as jnp
from jax import lax
from jax.experimental import pallas as pl
from jax.experimental.pallas import tpu as pltpu
from jax.experimental.pallas import tpu_sc as plsc

_N = 16384
_D = 128
_LOG2PI = float(np.log(2.0 * np.pi))

_NW = 32
_BPW = _N // _NW
_CHUNK = 128
_NCHUNK = _BPW // _CHUNK


def _rotl(x, r):
    return ((x << np.uint32(r)) | (x >> np.uint32(32 - r))).astype(np.uint32)


def _threefry2x32(k0, k1, x0, x1):
    k0 = np.uint32(k0)
    k1 = np.uint32(k1)
    ks2 = np.uint32(k0 ^ k1 ^ np.uint32(0x1BD11BDA))
    x0 = x0.astype(np.uint32).copy()
    x1 = x1.astype(np.uint32).copy()
    rot1 = (13, 15, 26, 6)
    rot2 = (17, 29, 16, 24)
    x0 += k0
    x1 += k1
    for r in rot1:
        x0 += x1
        x1 = _rotl(x1, r)
        x1 ^= x0
    x0 += k1
    x1 += ks2 + np.uint32(1)
    for r in rot2:
        x0 += x1
        x1 = _rotl(x1, r)
        x1 ^= x0
    x0 += ks2
    x1 += k0 + np.uint32(2)
    for r in rot1:
        x0 += x1
        x1 = _rotl(x1, r)
        x1 ^= x0
    x0 += k0
    x1 += k1 + np.uint32(3)
    for r in rot2:
        x0 += x1
        x1 = _rotl(x1, r)
        x1 ^= x0
    x0 += k1
    x1 += ks2 + np.uint32(4)
    for r in rot1:
        x0 += x1
        x1 = _rotl(x1, r)
        x1 ^= x0
    x0 += ks2
    x1 += k0 + np.uint32(5)
    return x0, x1


def _random_bits(k0, k1, n):
    i = np.arange(n, dtype=np.uint64)
    x0 = (i >> np.uint64(32)).astype(np.uint32)
    x1 = (i & np.uint64(0xFFFFFFFF)).astype(np.uint32)
    o0, o1 = _threefry2x32(k0, k1, x0, x1)
    return o0 ^ o1


def _bits_to_unit_float(bits):
    fb = (bits >> np.uint32(9)) | np.uint32(0x3F800000)
    return fb.view(np.float32) - np.float32(1.0)


def _erfinv64(x):
    w = -np.log((np.float64(1.0) - x) * (np.float64(1.0) + x))
    small = w < 5.0
    ws = w - 2.5
    p_s = np.float64(2.81022636e-08)
    for c in (3.43273939e-07, -3.5233877e-06, -4.39150654e-06, 0.00021858087,
              -0.00125372503, -0.00417768164, 0.246640727, 1.50140941):
        p_s = p_s * ws + c
    wl = np.sqrt(np.maximum(w, 1e-30)) - 3.0
    p_l = np.float64(-0.000200214257)
    for c in (0.000100950558, 0.00134934322, -0.00367342844, 0.00573950773,
              -0.0076224613, 0.00943887047, 1.00167406, 2.83297682):
        p_l = p_l * wl + c
    return np.where(small, p_s, p_l) * x


def _rng_constants():
    with np.errstate(over="ignore"):
        o0, o1 = _threefry2x32(np.uint32(0), np.uint32(42),
                               np.zeros(2, np.uint32),
                               np.arange(2, dtype=np.uint32))
        rk = (o0[0], o1[0])
        pk = (o0[1], o1[1])
        u0 = np.maximum(np.float32(0.0),
                        _bits_to_unit_float(_random_bits(rk[0], rk[1], 1)))[0]
        f = _bits_to_unit_float(_random_bits(pk[0], pk[1], _N * _D))
        lo = np.float32(np.nextafter(np.float32(-1.0), np.float32(0.0)))
        u = np.maximum(lo, (f * (np.float32(1.0) - lo) + lo).astype(np.float32))
        noise = (np.float64(np.sqrt(2.0))
                 * _erfinv64(u.astype(np.float64))).astype(np.float32)
    return np.float32(u0), noise.reshape(_N, _D)


_U0, _NOISE = _rng_constants()
import ml_dtypes as _ml_dtypes

_NOISE_BF16 = _NOISE.astype(_ml_dtypes.bfloat16)


def _sc_resample_body(cum_hbm, part_hbm, out_hbm, cum_v, idx_v, rows_v,
                      gsems, ssem):
    wid = lax.axis_index("s") * 2 + lax.axis_index("c")
    base = wid * _BPW
    pltpu.sync_copy(cum_hbm, cum_v)
    lane = lax.iota(jnp.int32, 16)
    u0 = jnp.float32(_U0)
    inv_n = jnp.float32(1.0 / _N)

    _ILP = 4

    def _rank16x(i_vecs):
        us = [(iv.astype(jnp.float32) + u0) * inv_n for iv in i_vecs]
        los = [jnp.zeros((16,), jnp.int32) for _ in i_vecs]
        half = _N // 2
        while half >= 1:
            probes = [lo + (half - 1) for lo in los]
            vals = [plsc.load_gather(cum_v, [p]) for p in probes]
            los = [lo + jnp.where(v < u, half, 0).astype(jnp.int32)
                   for lo, v, u in zip(los, vals, us)]
            half //= 2
        return [jnp.minimum(lo, _N - 1) for lo in los]

    gathers = []
    for j in range(_NCHUNK):
        row0 = base + j * _CHUNK

        def body(g, carry, row0=row0, j=j):
            col = g * (16 * _ILP)
            ranks = _rank16x([lane + (row0 + col + 16 * t)
                              for t in range(_ILP)])
            for t in range(_ILP):
                idx_v[j, pl.ds(col + 16 * t, 16)] = ranks[t]
            return carry

        lax.fori_loop(0, _CHUNK // (16 * _ILP), body, 0)
        gathers.append(
            pltpu.async_copy(part_hbm.at[idx_v.at[j]], rows_v.at[j], gsems[j]))
    stores = []
    for j in range(_NCHUNK):
        gathers[j].wait()
        stores.append(
            pltpu.async_copy(rows_v.at[j],
                             out_hbm.at[pl.ds(base + j * _CHUNK, _CHUNK)],
                             ssem))
    for s in stores:
        s.wait()


def _sc_resample(cum, particles):
    mesh = plsc.VectorSubcoreMesh(core_axis_name="c", subcore_axis_name="s")
    k = functools.partial(
        pl.kernel,
        mesh=mesh,
        out_type=jax.ShapeDtypeStruct((_N, _D), jnp.float32),
        scratch_types=[
            pltpu.VMEM((_N,), jnp.float32),
            pltpu.VMEM((_NCHUNK, _CHUNK), jnp.int32),
            pltpu.VMEM((_NCHUNK, _CHUNK, _D), jnp.float32),
            [pltpu.SemaphoreType.DMA] * _NCHUNK,
            pltpu.SemaphoreType.DMA,
        ],
        compiler_params=pltpu.CompilerParams(needs_layout_passes=False),
    )(_sc_resample_body)
    return k(cum, particles)


_BLK = 2048


def _dense_body(res_ref, noise_ref, logw_ref, lw_ref, obs_ref, A_ref, Ap_ref,
                b_ref, C_ref, pls_ref, qls_ref, ols_ref,
                nlw_ref, nxt_ref, ess_ref, acc_ref):
    i = pl.program_id(0)
    res = res_ref[...]
    noise = noise_ref[...].astype(jnp.float32)
    b = b_ref[...]
    pls = pls_ref[...]
    qls = qls_ref[...]
    ols = ols_ref[...]
    obs = obs_ref[...]

    dn = (((1,), (1,)), ((), ()))
    mean_p = lax.dot_general(res, Ap_ref[...], dn,
                             preferred_element_type=jnp.float32) + b
    nxt = mean_p + jnp.exp(qls) * noise
    mean_t = lax.dot_general(res, A_ref[...], dn,
                             preferred_element_type=jnp.float32) + b
    y = lax.dot_general(nxt, C_ref[...], dn,
                        preferred_element_type=jnp.float32)

    zt = (nxt - mean_t) * jnp.exp(-pls)
    zp = (nxt - mean_p) * jnp.exp(-qls)
    ze = (obs - y) * jnp.exp(-ols)

    half_d_log2pi = 0.5 * _D * _LOG2PI
    q = -0.5 * (zt * zt + ze * ze - zp * zp)
    const = jnp.sum(pls) + jnp.sum(ols) - jnp.sum(qls) + half_d_log2pi
    inc = (jnp.sum(q, axis=1, keepdims=True) - const).reshape(_BLK // _D, _D)
    nlw_ref[...] = logw_ref[...] + inc
    nxt_ref[...] = nxt

    lw = lw_ref[...]
    part = jnp.sum(jnp.exp(lw + lw), keepdims=True)

    @pl.when(i == 0)
    def _():
        acc_ref[...] = part

    @pl.when(i > 0)
    def _():
        acc_ref[...] = acc_ref[...] + part

    @pl.when(i == pl.num_programs(0) - 1)
    def _():
        ess_ref[...] = 1.0 / (jnp.float32(_N) * acc_ref[...])


def _dense(res, lw, log_w, observation, A, Ap, b, C, pls, qls, ols):
    grid = (_N // _BLK,)
    row_blk = lambda i: (i, 0)
    const_blk = lambda i: (0, 0)
    out_shapes = (
        jax.ShapeDtypeStruct((_N // _D, _D), jnp.float32),
        jax.ShapeDtypeStruct((_N, _D), jnp.float32),
        jax.ShapeDtypeStruct((1, 1), jnp.float32),
    )
    in_specs = [
        pl.BlockSpec((_BLK, _D), row_blk),
        pl.BlockSpec((_BLK, _D), row_blk),
        pl.BlockSpec((_BLK // _D, _D), row_blk),
        pl.BlockSpec((_BLK // _D, _D), row_blk),
        pl.BlockSpec((1, _D), const_blk),
        pl.BlockSpec((_D, _D), const_blk),
        pl.BlockSpec((_D, _D), const_blk),
        pl.BlockSpec((1, _D), const_blk),
        pl.BlockSpec((_D, _D), const_blk),
        pl.BlockSpec((1, _D), const_blk),
        pl.BlockSpec((1, _D), const_blk),
        pl.BlockSpec((1, _D), const_blk),
    ]
    out_specs = (
        pl.BlockSpec((_BLK // _D, _D), row_blk),
        pl.BlockSpec((_BLK, _D), row_blk),
        pl.BlockSpec((1, 1), const_blk),
    )
    return pl.pallas_call(
        _dense_body,
        grid=grid,
        in_specs=in_specs,
        out_specs=out_specs,
        out_shape=out_shapes,
        scratch_shapes=[pltpu.VMEM((1, 1), jnp.float32)],
        compiler_params=pltpu.CompilerParams(
            dimension_semantics=("arbitrary",),
        ),
    )(res, _NOISE_BF16, log_w.reshape(_N // _D, _D), lw.reshape(_N // _D, _D),
      observation.reshape(1, _D), A, Ap, b.reshape(1, _D), C,
      pls.reshape(1, _D), qls.reshape(1, _D), ols.reshape(1, _D))


def kernel(log_w, particles, observation, A, Ap, b, C,
           proc_log_scale, prop_log_scale, obs_log_scale):
    lw = log_w - jax.scipy.special.logsumexp(log_w)
    w = jnp.exp(lw)
    cum = jnp.cumsum(w)

    res = _sc_resample(cum, particles)
    nlw, nxt, ess = _dense(res, lw, log_w, observation, A, Ap, b, C,
                           proc_log_scale, prop_log_scale, obs_log_scale)
    return nlw.reshape(_N), nxt, ess.reshape(())


# speedup vs baseline: 9.4702x; 1.0107x over previous
import functools

import numpy as np

import jax
import jax.numpy as jnp
from jax import lax
from jax.experimental import pallas as pl
from jax.experimental.pallas import tpu as pltpu
from jax.experimental.pallas import tpu_sc as plsc

_N = 16384
_D = 128
_LOG2PI = float(np.log(2.0 * np.pi))

_NW = 32
_BPW = _N // _NW
_CHUNK = 128
_NCHUNK = _BPW // _CHUNK


def _rotl(x, r):
    return ((x << np.uint32(r)) | (x >> np.uint32(32 - r))).astype(np.uint32)


def _threefry2x32(k0, k1, x0, x1):
    k0 = np.uint32(k0)
    k1 = np.uint32(k1)
    ks2 = np.uint32(k0 ^ k1 ^ np.uint32(0x1BD11BDA))
    x0 = x0.astype(np.uint32).copy()
    x1 = x1.astype(np.uint32).copy()
    rot1 = (13, 15, 26, 6)
    rot2 = (17, 29, 16, 24)
    x0 += k0
    x1 += k1
    for r in rot1:
        x0 += x1
        x1 = _rotl(x1, r)
        x1 ^= x0
    x0 += k1
    x1 += ks2 + np.uint32(1)
    for r in rot2:
        x0 += x1
        x1 = _rotl(x1, r)
        x1 ^= x0
    x0 += ks2
    x1 += k0 + np.uint32(2)
    for r in rot1:
        x0 += x1
        x1 = _rotl(x1, r)
        x1 ^= x0
    x0 += k0
    x1 += k1 + np.uint32(3)
    for r in rot2:
        x0 += x1
        x1 = _rotl(x1, r)
        x1 ^= x0
    x0 += k1
    x1 += ks2 + np.uint32(4)
    for r in rot1:
        x0 += x1
        x1 = _rotl(x1, r)
        x1 ^= x0
    x0 += ks2
    x1 += k0 + np.uint32(5)
    return x0, x1


def _random_bits(k0, k1, n):
    i = np.arange(n, dtype=np.uint64)
    x0 = (i >> np.uint64(32)).astype(np.uint32)
    x1 = (i & np.uint64(0xFFFFFFFF)).astype(np.uint32)
    o0, o1 = _threefry2x32(k0, k1, x0, x1)
    return o0 ^ o1


def _bits_to_unit_float(bits):
    fb = (bits >> np.uint32(9)) | np.uint32(0x3F800000)
    return fb.view(np.float32) - np.float32(1.0)


def _erfinv64(x):
    w = -np.log((np.float64(1.0) - x) * (np.float64(1.0) + x))
    small = w < 5.0
    ws = w - 2.5
    p_s = np.float64(2.81022636e-08)
    for c in (3.43273939e-07, -3.5233877e-06, -4.39150654e-06, 0.00021858087,
              -0.00125372503, -0.00417768164, 0.246640727, 1.50140941):
        p_s = p_s * ws + c
    wl = np.sqrt(np.maximum(w, 1e-30)) - 3.0
    p_l = np.float64(-0.000200214257)
    for c in (0.000100950558, 0.00134934322, -0.00367342844, 0.00573950773,
              -0.0076224613, 0.00943887047, 1.00167406, 2.83297682):
        p_l = p_l * wl + c
    return np.where(small, p_s, p_l) * x


def _rng_constants():
    with np.errstate(over="ignore"):
        o0, o1 = _threefry2x32(np.uint32(0), np.uint32(42),
                               np.zeros(2, np.uint32),
                               np.arange(2, dtype=np.uint32))
        rk = (o0[0], o1[0])
        pk = (o0[1], o1[1])
        u0 = np.maximum(np.float32(0.0),
                        _bits_to_unit_float(_random_bits(rk[0], rk[1], 1)))[0]
        f = _bits_to_unit_float(_random_bits(pk[0], pk[1], _N * _D))
        lo = np.float32(np.nextafter(np.float32(-1.0), np.float32(0.0)))
        u = np.maximum(lo, (f * (np.float32(1.0) - lo) + lo).astype(np.float32))
        noise = (np.float64(np.sqrt(2.0))
                 * _erfinv64(u.astype(np.float64))).astype(np.float32)
    return np.float32(u0), noise.reshape(_N, _D)


_U0, _NOISE = _rng_constants()
import ml_dtypes as _ml_dtypes

_NOISE_BF16 = _NOISE.astype(_ml_dtypes.bfloat16)


def _sc_resample_body(cum_hbm, part_hbm, out_hbm, cum_v, idx_v, rows_v,
                      gsems, ssem):
    wid = lax.axis_index("s") * 2 + lax.axis_index("c")
    base = wid * _BPW
    pltpu.sync_copy(cum_hbm, cum_v)
    lane = lax.iota(jnp.int32, 16)
    u0 = jnp.float32(_U0)
    inv_n = jnp.float32(1.0 / _N)

    _ILP = 4

    def _rank16x(i_vecs):
        us = [(iv.astype(jnp.float32) + u0) * inv_n for iv in i_vecs]
        los = [jnp.zeros((16,), jnp.int32) for _ in i_vecs]
        half = _N // 2
        while half >= 1:
            probes = [lo + (half - 1) for lo in los]
            vals = [plsc.load_gather(cum_v, [p]) for p in probes]
            los = [lo + jnp.where(v < u, half, 0).astype(jnp.int32)
                   for lo, v, u in zip(los, vals, us)]
            half //= 2
        return [jnp.minimum(lo, _N - 1) for lo in los]

    gathers = []
    for j in range(_NCHUNK):
        row0 = base + j * _CHUNK

        def body(g, carry, row0=row0, j=j):
            col = g * (16 * _ILP)
            ranks = _rank16x([lane + (row0 + col + 16 * t)
                              for t in range(_ILP)])
            for t in range(_ILP):
                idx_v[j, pl.ds(col + 16 * t, 16)] = ranks[t]
            return carry

        lax.fori_loop(0, _CHUNK // (16 * _ILP), body, 0)
        gathers.append(
            pltpu.async_copy(part_hbm.at[idx_v.at[j]], rows_v.at[j], gsems[j]))
    stores = []
    for j in range(_NCHUNK):
        gathers[j].wait()
        stores.append(
            pltpu.async_copy(rows_v.at[j],
                             out_hbm.at[pl.ds(base + j * _CHUNK, _CHUNK)],
                             ssem))
    for s in stores:
        s.wait()


def _sc_resample(cum, particles):
    mesh = plsc.VectorSubcoreMesh(core_axis_name="c", subcore_axis_name="s")
    k = functools.partial(
        pl.kernel,
        mesh=mesh,
        out_type=jax.ShapeDtypeStruct((_N, _D), jnp.float32),
        scratch_types=[
            pltpu.VMEM((_N,), jnp.float32),
            pltpu.VMEM((_NCHUNK, _CHUNK), jnp.int32),
            pltpu.VMEM((_NCHUNK, _CHUNK, _D), jnp.float32),
            [pltpu.SemaphoreType.DMA] * _NCHUNK,
            pltpu.SemaphoreType.DMA,
        ],
        compiler_params=pltpu.CompilerParams(
            needs_layout_passes=False,
            disable_bounds_checks=True,
            disable_semaphore_checks=True,
            skip_device_barrier=True,
        ),
    )(_sc_resample_body)
    return k(cum, particles)


_BLK = 2048


def _dense_body(res_ref, noise_ref, logw_ref, lw_ref, obs_ref, A_ref, Ap_ref,
                b_ref, C_ref, pls_ref, qls_ref, ols_ref,
                nlw_ref, nxt_ref, ess_ref, acc_ref):
    i = pl.program_id(0)
    res = res_ref[...]
    noise = noise_ref[...].astype(jnp.float32)
    b = b_ref[...]
    pls = pls_ref[...]
    qls = qls_ref[...]
    ols = ols_ref[...]
    obs = obs_ref[...]

    dn = (((1,), (1,)), ((), ()))
    mean_p = lax.dot_general(res, Ap_ref[...], dn,
                             preferred_element_type=jnp.float32) + b
    nxt = mean_p + jnp.exp(qls) * noise
    mean_t = lax.dot_general(res, A_ref[...], dn,
                             preferred_element_type=jnp.float32) + b
    y = lax.dot_general(nxt, C_ref[...], dn,
                        preferred_element_type=jnp.float32)

    zt = (nxt - mean_t) * jnp.exp(-pls)
    zp = (nxt - mean_p) * jnp.exp(-qls)
    ze = (obs - y) * jnp.exp(-ols)

    half_d_log2pi = 0.5 * _D * _LOG2PI
    q = -0.5 * (zt * zt + ze * ze - zp * zp)
    const = jnp.sum(pls) + jnp.sum(ols) - jnp.sum(qls) + half_d_log2pi
    inc = (jnp.sum(q, axis=1, keepdims=True) - const).reshape(_BLK // _D, _D)
    nlw_ref[...] = logw_ref[...] + inc
    nxt_ref[...] = nxt

    lw = lw_ref[...]
    part = jnp.sum(jnp.exp(lw + lw), keepdims=True)

    @pl.when(i == 0)
    def _():
        acc_ref[...] = part

    @pl.when(i > 0)
    def _():
        acc_ref[...] = acc_ref[...] + part

    @pl.when(i == pl.num_programs(0) - 1)
    def _():
        ess_ref[...] = 1.0 / (jnp.float32(_N) * acc_ref[...])


def _dense(res, lw, log_w, observation, A, Ap, b, C, pls, qls, ols):
    grid = (_N // _BLK,)
    row_blk = lambda i: (i, 0)
    const_blk = lambda i: (0, 0)
    out_shapes = (
        jax.ShapeDtypeStruct((_N // _D, _D), jnp.float32),
        jax.ShapeDtypeStruct((_N, _D), jnp.float32),
        jax.ShapeDtypeStruct((1, 1), jnp.float32),
    )
    in_specs = [
        pl.BlockSpec((_BLK, _D), row_blk),
        pl.BlockSpec((_BLK, _D), row_blk),
        pl.BlockSpec((_BLK // _D, _D), row_blk),
        pl.BlockSpec((_BLK // _D, _D), row_blk),
        pl.BlockSpec((1, _D), const_blk),
        pl.BlockSpec((_D, _D), const_blk),
        pl.BlockSpec((_D, _D), const_blk),
        pl.BlockSpec((1, _D), const_blk),
        pl.BlockSpec((_D, _D), const_blk),
        pl.BlockSpec((1, _D), const_blk),
        pl.BlockSpec((1, _D), const_blk),
        pl.BlockSpec((1, _D), const_blk),
    ]
    out_specs = (
        pl.BlockSpec((_BLK // _D, _D), row_blk),
        pl.BlockSpec((_BLK, _D), row_blk),
        pl.BlockSpec((1, 1), const_blk),
    )
    return pl.pallas_call(
        _dense_body,
        grid=grid,
        in_specs=in_specs,
        out_specs=out_specs,
        out_shape=out_shapes,
        scratch_shapes=[pltpu.VMEM((1, 1), jnp.float32)],
        compiler_params=pltpu.CompilerParams(
            dimension_semantics=("arbitrary",),
        ),
    )(res, _NOISE_BF16, log_w.reshape(_N // _D, _D), lw.reshape(_N // _D, _D),
      observation.reshape(1, _D), A, Ap, b.reshape(1, _D), C,
      pls.reshape(1, _D), qls.reshape(1, _D), ols.reshape(1, _D))


def kernel(log_w, particles, observation, A, Ap, b, C,
           proc_log_scale, prop_log_scale, obs_log_scale):
    lw = log_w - jax.scipy.special.logsumexp(log_w)
    w = jnp.exp(lw)
    cum = jnp.cumsum(w)

    res = _sc_resample(cum, particles)
    nlw, nxt, ess = _dense(res, lw, log_w, observation, A, Ap, b, C,
                           proc_log_scale, prop_log_scale, obs_log_scale)
    return nlw.reshape(_N), nxt, ess.reshape(())
